# Initial kernel scaffold; baseline (speedup 1.0000x reference)
import jax
import jax.numpy as jnp
from jax.experimental import pallas as pl


def kernel(x1, edge_index1, x2, edge_index2, W_init, b_init, W_att, b_att, W_fc, b_fc, W_fc2, b_fc2, W_fc3, b_fc3):
    raise NotImplementedError("write your pallas kernel here")


# speedup vs baseline: 8.5609x; 8.5609x over previous
import functools

import jax
import jax.numpy as jnp
from jax import lax
from jax.experimental import pallas as pl
from jax.experimental.pallas import tpu as pltpu
from jax.experimental.pallas import tpu_sc as plsc

_N = 1024
_E = 32768
_TBL = 2 * _N
_D = 128
_NC = 2
_NS = 16
_NW = _NC * _NS
_EPW = 2 * _E // _NW
_CH = 128
_NCH = _EPW // _CH


def _pre_body(xb_ref, wext_ref, brow_ref, out_ref):
    dn = (((1,), (1,)), ((), ()))
    h = lax.dot_general(xb_ref[...], wext_ref[...], dn,
                        preferred_element_type=jnp.float32)
    out_ref[...] = h + brow_ref[...]


def _pre(xb, wext, brow):
    return pl.pallas_call(
        _pre_body,
        out_shape=jax.ShapeDtypeStruct((_TBL, _D), jnp.float32),
    )(xb, wext, brow)


def _sc_body(tbl_hbm, src_hbm, dst_hbm, z_hbm, out_hbm,
             idx_s, idx_d, rows, acc, sem):
    c = lax.axis_index("c")
    s = lax.axis_index("s")
    w = s * _NC + c
    rps = _TBL // _NS
    pltpu.sync_copy(z_hbm.at[pl.ds(s * rps, rps)], acc.at[pl.ds(s * rps, rps)])
    plsc.subcore_barrier()
    pltpu.sync_copy(src_hbm.at[pl.ds(w * _NCH, _NCH)], idx_s)
    pltpu.sync_copy(dst_hbm.at[pl.ds(w * _NCH, _NCH)], idx_d)
    for j in range(_NCH):
        pltpu.async_copy(tbl_hbm.at[idx_s.at[j]], rows, sem).wait()
        pltpu.sync_copy(rows, acc.at[idx_d.at[j]], add=True)
    plsc.subcore_barrier()
    pltpu.sync_copy(acc.at[pl.ds(s * rps, rps)],
                    out_hbm.at[c, pl.ds(s * rps, rps)])


@functools.lru_cache(maxsize=None)
def _get_sc_agg():
    return pl.kernel(
        _sc_body,
        out_type=jax.ShapeDtypeStruct((_NC, _TBL, _D), jnp.float32),
        mesh=plsc.VectorSubcoreMesh(core_axis_name="c", subcore_axis_name="s",
                                    num_cores=_NC, num_subcores=_NS),
        scratch_types=[
            pltpu.VMEM((_NCH, _CH), jnp.int32),
            pltpu.VMEM((_NCH, _CH), jnp.int32),
            pltpu.VMEM((_CH, _D), jnp.float32),
            pltpu.VMEM_SHARED((_TBL, _D), jnp.float32),
            pltpu.SemaphoreType.DMA,
        ],
    )


def _two_sum(a, b):
    s = a + b
    t = s - a
    e = (a - (s - t)) + (b - t)
    return s, e


def _head_body(parts_ref, wa1_ref, wa2_ref, batt_ref,
               wf1_ref, wf2_ref, bfc_ref, wfc2_ref, bfc2_ref,
               wfc3_ref, bfc3_ref, out_ref):
    f32 = jnp.float32
    dnT = (((1,), (1,)), ((), ()))
    nf = f32(_N)

    a = parts_ref[0] + parts_ref[1]
    col = lax.broadcasted_iota(jnp.int32, (_TBL, _D), 1)
    cnt = (col == 32).astype(f32)
    deg = jnp.sum(a * cnt, axis=1, keepdims=True)
    t = a / jnp.maximum(deg, 1.0)
    t1 = t[:_N]
    t2 = t[_N:]

    u = lax.dot_general(t1, wa1_ref[...], dnT, preferred_element_type=f32)
    v = lax.dot_general(t2, wa2_ref[...], dnT, preferred_element_type=f32)
    bb = batt_ref[0, 0]
    Su = jnp.sum(u)
    Sv = jnp.sum(v)
    T1 = jnp.sum(t1, axis=0, keepdims=True)
    T2 = jnp.sum(t2, axis=0, keepdims=True)
    Ut1 = jnp.sum(u * t1, axis=0, keepdims=True)
    Vt2 = jnp.sum(v * t2, axis=0, keepdims=True)

    def side(w_, S_other):
        sh, se = _two_sum(w_, bb)
        h2, e2 = _two_sum(nf * sh, S_other)
        lo = nf * se + e2
        rcp = 1.0 / h2
        r = rcp - (lo * rcp) * rcp
        A = jnp.sum(sh * r)
        B = jnp.sum(r)
        return A, B

    A1, B1 = side(u, Sv)
    A2, B2 = side(v, Su)
    g1 = (T1 + A1 * T2 + B1 * Vt2) * (1.0 / nf)
    g2 = (T2 + A2 * T1 + B2 * Ut1) * (1.0 / nf)

    H = (lax.dot_general(g1, wf1_ref[...], dnT, preferred_element_type=f32)
         + lax.dot_general(g2, wf2_ref[...], dnT, preferred_element_type=f32)
         + bfc_ref[...])
    H2 = lax.dot_general(H, wfc2_ref[...], dnT,
                         preferred_element_type=f32) + bfc2_ref[...]
    hb = H2.astype(jnp.bfloat16).astype(f32)
    wb = wfc3_ref[...].astype(jnp.bfloat16).astype(f32)
    out_ref[0, 0] = jnp.sum(hb * wb) + bfc3_ref[0, 0]


def _head(parts, wa1, wa2, batt, wf1, wf2, bfc, wfc2, bfc2, wfc3, bfc3):
    vmem = pl.BlockSpec(memory_space=pltpu.VMEM)
    smem = pl.BlockSpec(memory_space=pltpu.SMEM)
    return pl.pallas_call(
        _head_body,
        in_specs=[vmem, vmem, vmem, smem, vmem, vmem, vmem, vmem, vmem,
                  vmem, smem],
        out_specs=smem,
        out_shape=jax.ShapeDtypeStruct((1, 1), jnp.float32),
    )(parts, wa1, wa2, batt, wf1, wf2, bfc, wfc2, bfc2, wfc3, bfc3)


def kernel(x1, edge_index1, x2, edge_index2, W_init, b_init, W_att, b_att,
           W_fc, b_fc, W_fc2, b_fc2, W_fc3, b_fc3):
    f32 = jnp.float32
    xb = jnp.concatenate([x1, x2], axis=0)

    d_in = x1.shape[1]
    wext = jnp.zeros((_D, d_in), f32)
    wext = wext.at[:32, :].set(W_init)
    brow = jnp.zeros((1, _D), f32)
    brow = brow.at[0, :32].set(b_init)
    brow = brow.at[0, 32].set(1.0)

    tbl = _pre(xb, wext, brow)

    src = jnp.concatenate([edge_index1[0], edge_index2[0] + _N])
    dst = jnp.concatenate([edge_index1[1], edge_index2[1] + _N])
    src = src.reshape(_NW * _NCH, _CH).astype(jnp.int32)
    dst = dst.reshape(_NW * _NCH, _CH).astype(jnp.int32)
    z = jnp.zeros((_TBL, _D), f32)

    parts = _get_sc_agg()(tbl, src, dst, z)

    pad1 = jnp.zeros((1, _D - 32), f32)
    wa1 = jnp.concatenate([W_att[:, :32], pad1], axis=1)
    wa2 = jnp.concatenate([W_att[:, 32:], pad1], axis=1)
    batt = b_att.reshape(1, 1)
    padw = jnp.zeros((256, _D - 32), f32)
    wf1 = jnp.concatenate([W_fc[:, :32], padw], axis=1)
    wf2 = jnp.concatenate([W_fc[:, 32:], padw], axis=1)
    return _head(parts, wa1, wa2, batt, wf1, wf2, b_fc.reshape(1, 256),
                 W_fc2, b_fc2.reshape(1, 256), W_fc3, b_fc3.reshape(1, 1))
